# Initial kernel scaffold; baseline (speedup 1.0000x reference)
import jax
import jax.numpy as jnp
from jax.experimental import pallas as pl


def kernel(iword, oword, nword, refs, W_in, W_out):
    raise NotImplementedError("write your pallas kernel here")


# speedup vs baseline: 1.1029x; 1.1029x over previous
import functools

import jax
import jax.numpy as jnp
from jax import lax
from jax.experimental import pallas as pl
from jax.experimental.pallas import tpu as pltpu
from jax.experimental.pallas import tpu_sc as plsc


def _make_sc_kernel(B, N, D, V, C):
  info = plsc.get_sparse_core_info()
  NC, NS = info.num_cores, info.num_subcores
  NW = NC * NS
  assert B % NW == 0
  b_per_w = B // NW
  assert b_per_w % C == 0
  n_chunks = b_per_w // C
  CN = C * N
  assert CN % 16 == 0
  G = CN // 16
  P = 128
  n_p = (CN + P - 1) // P
  assert CN % P == 0
  DV = D // 16

  mesh = plsc.VectorSubcoreMesh(core_axis_name="c", subcore_axis_name="s")

  @functools.partial(
      pl.kernel,
      mesh=mesh,
      compiler_params=pltpu.CompilerParams(
          use_tc_tiling_on_sc=False, needs_layout_passes=False),
      out_type=[jax.ShapeDtypeStruct((B, D), jnp.float32)] * 3,
      scratch_types=[
          pltpu.VMEM((C,), jnp.int32),
          pltpu.VMEM((C, N), jnp.int32),
          pltpu.VMEM((CN,), jnp.int32),
          pltpu.VMEM((CN, D), jnp.float32),
          pltpu.VMEM((C, D), jnp.float32),
          pltpu.SemaphoreType.DMA,
      ],
  )
  def sc_kernel(iw_h, ow_h, nw_h, refs_h, win_h, wout_h,
                iout_h, oout_h, nout_h,
                widx_v, r2_v, rflat_v, rows_v, outc_v, sem):
    wid = lax.axis_index("s") * NC + lax.axis_index("c")
    base = wid * b_per_w
    iota = lax.iota(jnp.int32, 16)
    inv_n = jnp.float32(1.0 / N)

    for src_h, tbl_h, dst_h in ((iw_h, win_h, iout_h),
                                (ow_h, wout_h, oout_h),
                                (nw_h, wout_h, nout_h)):
      def chunk_body(c, carry, src_h=src_h, tbl_h=tbl_h, dst_h=dst_h):
        off = pl.multiple_of(base + c * C, C)
        pltpu.sync_copy(src_h.at[pl.ds(off, C)], widx_v)
        pltpu.async_copy(refs_h.at[widx_v], r2_v, sem).wait()
        def fl_body(g, carry2):
          kk = iota + g * 16
          w = kk // N
          j = kk - w * N
          vals = plsc.load_gather(r2_v, [w, j])
          rflat_v[pl.ds(pl.multiple_of(g * 16, 16), 16)] = vals
          return carry2
        lax.fori_loop(0, G, fl_body, 0)
        descs = [
            pltpu.async_copy(
                tbl_h.at[rflat_v.at[pl.ds(p * P, P)]],
                rows_v.at[pl.ds(p * P, P)],
                sem,
            )
            for p in range(n_p)
        ]
        for d in descs:
          d.wait()
        def w_body(w, carry2):
          rb = w * N
          def r_body(r, accs):
            return tuple(
                accs[d] + rows_v[rb + r, pl.ds(d * 16, 16)]
                for d in range(DV)
            )
          accs = lax.fori_loop(
              0, N, r_body,
              tuple(jnp.zeros((16,), jnp.float32) for _ in range(DV)))
          for d in range(DV):
            outc_v[w, pl.ds(d * 16, 16)] = accs[d] * inv_n
          return carry2
        lax.fori_loop(0, C, w_body, 0)
        pltpu.sync_copy(outc_v, dst_h.at[pl.ds(off, C)])
        return carry
      lax.fori_loop(0, n_chunks, chunk_body, 0)

  return sc_kernel


def _loss_tc(B, D, ivec, ovec, nvec):
  def body(iv_ref, ov_ref, nv_ref, out_ref):
    iv = iv_ref[...]
    ov = ov_ref[...]
    nv = nv_ref[...]
    po = jnp.sum(iv * ov, axis=1)
    pn = jnp.sum(iv * nv, axis=1)

    def log_sig(x):
      return jnp.minimum(x, 0.0) - jnp.log1p(jnp.exp(-jnp.abs(x)))

    loss = -(log_sig(po) + log_sig(-pn))
    out_ref[0, 0] = jnp.sum(loss) * (1.0 / B)

  out = pl.pallas_call(
      body,
      out_shape=jax.ShapeDtypeStruct((1, 1), jnp.float32),
      out_specs=pl.BlockSpec(memory_space=pltpu.SMEM),
  )(ivec, ovec, nvec)
  return out[0, 0]


def kernel(iword, oword, nword, refs, W_in, W_out):
  B = iword.shape[0]
  N = refs.shape[1]
  V, D = W_in.shape
  iword = iword.astype(jnp.int32)
  oword = oword.astype(jnp.int32)
  nword = nword.astype(jnp.int32)
  sc = _make_sc_kernel(B, N, D, V, 64)
  ivec, ovec, nvec = sc(iword, oword, nword, refs, W_in, W_out)
  return _loss_tc(B, D, ivec, ovec, nvec)


# speedup vs baseline: 1.1031x; 1.0002x over previous
import functools

import jax
import jax.numpy as jnp
from jax import lax
from jax.experimental import pallas as pl
from jax.experimental.pallas import tpu as pltpu
from jax.experimental.pallas import tpu_sc as plsc


def _make_sc_kernel(B, N, D, V, C):
  info = plsc.get_sparse_core_info()
  NC, NS = info.num_cores, info.num_subcores
  NW = NC * NS
  assert B % NW == 0
  b_per_w = B // NW
  assert b_per_w % C == 0
  n_chunks = b_per_w // C
  CN = C * N
  assert CN % 16 == 0
  G = CN // 16
  P = 128
  n_p = (CN + P - 1) // P
  assert CN % P == 0
  DV = D // 16

  mesh = plsc.VectorSubcoreMesh(core_axis_name="c", subcore_axis_name="s")

  @functools.partial(
      pl.kernel,
      mesh=mesh,
      compiler_params=pltpu.CompilerParams(
          use_tc_tiling_on_sc=False, needs_layout_passes=False),
      out_type=[jax.ShapeDtypeStruct((B, D), jnp.float32)] * 3,
      scratch_types=[
          pltpu.VMEM((C,), jnp.int32),
          pltpu.VMEM((C, N), jnp.int32),
          pltpu.VMEM((CN,), jnp.int32),
          pltpu.VMEM((CN, D), jnp.float32),
          pltpu.VMEM((C, D), jnp.float32),
          pltpu.SemaphoreType.DMA,
      ],
  )
  def sc_kernel(iw_h, ow_h, nw_h, refs_h, win_h, wout_h,
                iout_h, oout_h, nout_h,
                widx_v, r2_v, rflat_v, rows_v, outc_v, sem):
    wid = lax.axis_index("s") * NC + lax.axis_index("c")
    base = wid * b_per_w
    iota = lax.iota(jnp.int32, 16)
    inv_n = jnp.float32(1.0 / N)

    for src_h, tbl_h, dst_h in ((iw_h, win_h, iout_h),
                                (ow_h, wout_h, oout_h),
                                (nw_h, wout_h, nout_h)):
      def chunk_body(c, carry, src_h=src_h, tbl_h=tbl_h, dst_h=dst_h):
        off = pl.multiple_of(base + c * C, C)
        pltpu.sync_copy(src_h.at[pl.ds(off, C)], widx_v)
        pltpu.async_copy(refs_h.at[widx_v], r2_v, sem).wait()
        def fl_body(g, carry2):
          kk = iota + g * 16
          w = kk // N
          j = kk - w * N
          vals = plsc.load_gather(r2_v, [w, j])
          rflat_v[pl.ds(pl.multiple_of(g * 16, 16), 16)] = vals
          return carry2
        lax.fori_loop(0, G, fl_body, 0)
        descs = [
            pltpu.async_copy(
                tbl_h.at[rflat_v.at[pl.ds(p * P, P)]],
                rows_v.at[pl.ds(p * P, P)],
                sem,
            )
            for p in range(n_p)
        ]
        for d in descs:
          d.wait()
        def w_body(w, carry2):
          rb = w * N
          accs = [rows_v[rb, pl.ds(d * 16, 16)] for d in range(DV)]
          for r in range(1, N):
            for d in range(DV):
              accs[d] = accs[d] + rows_v[rb + r, pl.ds(d * 16, 16)]
          for d in range(DV):
            outc_v[w, pl.ds(d * 16, 16)] = accs[d] * inv_n
          return carry2
        lax.fori_loop(0, C, w_body, 0)
        pltpu.sync_copy(outc_v, dst_h.at[pl.ds(off, C)])
        return carry
      lax.fori_loop(0, n_chunks, chunk_body, 0)

  return sc_kernel


def _loss_tc(B, D, ivec, ovec, nvec):
  def body(iv_ref, ov_ref, nv_ref, out_ref):
    iv = iv_ref[...]
    ov = ov_ref[...]
    nv = nv_ref[...]
    po = jnp.sum(iv * ov, axis=1)
    pn = jnp.sum(iv * nv, axis=1)

    def log_sig(x):
      return jnp.minimum(x, 0.0) - jnp.log1p(jnp.exp(-jnp.abs(x)))

    loss = -(log_sig(po) + log_sig(-pn))
    out_ref[0, 0] = jnp.sum(loss) * (1.0 / B)

  out = pl.pallas_call(
      body,
      out_shape=jax.ShapeDtypeStruct((1, 1), jnp.float32),
      out_specs=pl.BlockSpec(memory_space=pltpu.SMEM),
  )(ivec, ovec, nvec)
  return out[0, 0]


def kernel(iword, oword, nword, refs, W_in, W_out):
  B = iword.shape[0]
  N = refs.shape[1]
  V, D = W_in.shape
  iword = iword.astype(jnp.int32)
  oword = oword.astype(jnp.int32)
  nword = nword.astype(jnp.int32)
  sc = _make_sc_kernel(B, N, D, V, 64)
  ivec, ovec, nvec = sc(iword, oword, nword, refs, W_in, W_out)
  return _loss_tc(B, D, ivec, ovec, nvec)
